# Initial kernel scaffold; baseline (speedup 1.0000x reference)
import functools

import jax
import jax.numpy as jnp
from jax import lax
from jax.experimental import pallas as pl
from jax.experimental.pallas import tpu as pltpu
from jax.experimental.pallas import tpu_sc as plsc

N = 10000
E = 320000
F_IN = 128
H = 30
P = 10
C = 2

NC, NS, L = 2, 16, 16
NW = NC * NS
N_PAD = 10240
HP = 32
E_PAD = 327680
EPW = E_PAD // NW
CH = 128
NSTREAM = EPW // CH
RPT = N_PAD // NS

_MESH = plsc.VectorSubcoreMesh(
    core_axis_name="c", subcore_axis_name="s", num_cores=NC, num_subcores=NS
)


@functools.partial(
    pl.kernel,
    out_type=jax.ShapeDtypeStruct((NW, N_PAD), jnp.float32),
    mesh=_MESH,
    scratch_types=[
        pltpu.VMEM((EPW,), jnp.int32),
        pltpu.VMEM((N_PAD,), jnp.float32),
    ],
)
def _deg_sc(col_hbm, out_hbm, col_v, deg_v):
    cc = lax.axis_index("c")
    ss = lax.axis_index("s")
    wid = ss * NC + cc
    pltpu.sync_copy(col_hbm.at[wid], col_v)

    zero16 = jnp.zeros((L,), jnp.float32)

    def zbody(i, carry):
        deg_v[pl.ds(i * L, L)] = zero16
        return carry

    lax.fori_loop(0, N_PAD // L, zbody, 0, unroll=8)

    ones16 = jnp.ones((L,), jnp.float32)

    def body(i, carry):
        idx = col_v[pl.ds(i * L, L)]
        plsc.addupdate_scatter(deg_v, [idx], ones16)
        return carry

    lax.fori_loop(0, EPW // L, body, 0, unroll=8)
    pltpu.sync_copy(deg_v, out_hbm.at[wid])


@functools.partial(
    pl.kernel,
    out_type=jax.ShapeDtypeStruct((NC, N_PAD, HP), jnp.float32),
    mesh=_MESH,
    scratch_types=[
        pltpu.VMEM((NSTREAM, CH), jnp.int32),
        pltpu.VMEM((NSTREAM, CH), jnp.int32),
        pltpu.VMEM((CH, HP), jnp.float32),
        pltpu.VMEM_SHARED((N_PAD, HP), jnp.float32),
        pltpu.SemaphoreType.DMA,
    ],
)
def _agg_sc(z_hbm, row_hbm, col_hbm, zrs_hbm, out_hbm, row_v, col_v, buf, acc_sh, sem):
    cc = lax.axis_index("c")
    ss = lax.axis_index("s")
    wid = ss * NC + cc

    pltpu.sync_copy(zrs_hbm.at[pl.ds(ss * RPT, RPT)], acc_sh.at[pl.ds(ss * RPT, RPT)])
    pltpu.sync_copy(row_hbm.at[wid], row_v)
    pltpu.sync_copy(col_hbm.at[wid], col_v)
    plsc.subcore_barrier()

    def body(j, carry):
        pltpu.async_copy(z_hbm.at[row_v.at[j]], buf, sem).wait()
        pltpu.sync_copy(buf, acc_sh.at[col_v.at[j]], add=True)
        return carry

    lax.fori_loop(0, NSTREAM, body, 0)

    plsc.subcore_barrier()
    pltpu.sync_copy(
        acc_sh.at[pl.ds(ss * RPT, RPT)], out_hbm.at[cc, pl.ds(ss * RPT, RPT)]
    )


def _tc1_body(degp, x, w1, z1, dinv):
    ones = jnp.ones((NW, 1), jnp.float32)
    deg = lax.dot_general(
        degp[...], ones, (((0,), (0,)), ((), ())),
        preferred_element_type=jnp.float32,
    ) + 1.0
    di = lax.rsqrt(deg)
    h = jnp.dot(x[...], w1[...], preferred_element_type=jnp.float32)
    z1[...] = di * h
    dinv[...] = di


def _tc_mid_body(aggp, z, dinv, b, w, zout):
    di = dinv[...]
    xn = jnp.maximum(di * (aggp[0] + aggp[1] + z[...]) + b[...], 0.0)
    h = jnp.dot(xn, w[...], preferred_element_type=jnp.float32)
    zout[...] = di * h


def _tc4_body(aggp, z, dinv, b, lw1, lb1, lw2, lb2, out):
    di = dinv[...]
    xn = jnp.maximum(di * (aggp[0] + aggp[1] + z[...]) + b[...], 0.0)
    ridx = lax.broadcasted_iota(jnp.int32, (N_PAD, HP), 0)
    xn = jnp.where(ridx < N, xn, -jnp.inf)
    g = jnp.max(xn, axis=0, keepdims=True)
    o1 = jnp.maximum(
        jnp.dot(g, lw1[...], preferred_element_type=jnp.float32) + lb1[...], 0.0
    )
    out[...] = jnp.dot(o1, lw2[...], preferred_element_type=jnp.float32) + lb2[...]


_tc1 = pl.pallas_call(
    _tc1_body,
    out_shape=[
        jax.ShapeDtypeStruct((N_PAD, HP), jnp.float32),
        jax.ShapeDtypeStruct((N_PAD, 1), jnp.float32),
    ],
)

_tc_mid = pl.pallas_call(
    _tc_mid_body,
    out_shape=jax.ShapeDtypeStruct((N_PAD, HP), jnp.float32),
)

_tc4 = pl.pallas_call(
    _tc4_body,
    out_shape=jax.ShapeDtypeStruct((1, 128), jnp.float32),
)


def kernel(x, edge_index, W1, b1, W2, b2, W3, b3, lW1, lb1, lW2, lb2):
    f32 = jnp.float32
    row = edge_index[0].astype(jnp.int32)
    col = edge_index[1].astype(jnp.int32)
    pad_e = E_PAD - E
    row_p = jnp.concatenate([row, jnp.full((pad_e,), N_PAD - 2, jnp.int32)])
    col_p = jnp.concatenate([col, jnp.full((pad_e,), N_PAD - 1, jnp.int32)])
    row3 = row_p.reshape(NW, NSTREAM, CH)
    col3 = col_p.reshape(NW, NSTREAM, CH)
    col2 = col_p.reshape(NW, EPW)

    x_pad = jnp.pad(x.astype(f32), ((0, N_PAD - N), (0, 0)))
    W1p = jnp.pad(W1, ((0, 0), (0, HP - H)))
    W2p = jnp.pad(W2, ((0, HP - H), (0, HP - H)))
    W3p = jnp.pad(W3, ((0, HP - H), (0, HP - H)))
    b1p = jnp.pad(b1, (0, HP - H)).reshape(1, HP)
    b2p = jnp.pad(b2, (0, HP - H)).reshape(1, HP)
    b3p = jnp.pad(b3, (0, HP - H)).reshape(1, HP)
    lW1p = jnp.pad(lW1, ((0, HP - H), (0, 128 - P)))
    lb1p = jnp.pad(lb1, (0, 128 - P)).reshape(1, 128)
    lW2p = jnp.pad(lW2, ((0, 128 - P), (0, 128 - C)))
    lb2p = jnp.pad(lb2, (0, 128 - C)).reshape(1, 128)
    zrs = jnp.zeros((N_PAD, HP), f32)

    degp = _deg_sc(col2)
    z1, dinv = _tc1(degp, x_pad, W1p)
    agg1 = _agg_sc(z1, row3, col3, zrs)
    z2 = _tc_mid(agg1, z1, dinv, b1p, W2p)
    agg2 = _agg_sc(z2, row3, col3, zrs)
    z3 = _tc_mid(agg2, z2, dinv, b2p, W3p)
    agg3 = _agg_sc(z3, row3, col3, zrs)
    outp = _tc4(agg3, z3, dinv, b3p, lW1p, lb1p, lW2p, lb2p)
    return outp[:, :C]


# speedup vs baseline: 17.8756x; 17.8756x over previous
import functools

import jax
import jax.numpy as jnp
from jax import lax
from jax.experimental import pallas as pl
from jax.experimental.pallas import tpu as pltpu
from jax.experimental.pallas import tpu_sc as plsc

N = 10000
E = 320000
F_IN = 128
H = 30
P = 10
C = 2

NC, NS, L = 2, 16, 16
NW = NC * NS
N_PAD = 10240
HP = 32
E_PAD = 327680
EPW = E_PAD // NW
CH = 128
NSTREAM = EPW // CH
RPT = N_PAD // NS

_MESH = plsc.VectorSubcoreMesh(
    core_axis_name="c", subcore_axis_name="s", num_cores=NC, num_subcores=NS
)


DW = 8


@functools.partial(
    pl.kernel,
    out_type=jax.ShapeDtypeStruct((NC, N_PAD, DW), jnp.float32),
    mesh=_MESH,
    scratch_types=[
        pltpu.VMEM((NSTREAM, CH), jnp.int32),
        pltpu.VMEM((CH, DW), jnp.float32),
        pltpu.VMEM_SHARED((N_PAD, DW), jnp.float32),
    ],
    compiler_params=pltpu.CompilerParams(use_tc_tiling_on_sc=False),
)
def _deg_sc(col_hbm, ones_hbm, zrs_hbm, out_hbm, col_v, ones_v, acc_sh):
    cc = lax.axis_index("c")
    ss = lax.axis_index("s")
    wid = ss * NC + cc

    pltpu.sync_copy(zrs_hbm.at[pl.ds(ss * RPT, RPT)], acc_sh.at[pl.ds(ss * RPT, RPT)])
    pltpu.sync_copy(ones_hbm, ones_v)
    pltpu.sync_copy(col_hbm.at[wid], col_v)
    plsc.subcore_barrier()

    def body(j, carry):
        pltpu.sync_copy(ones_v, acc_sh.at[col_v.at[j]], add=True)
        return carry

    lax.fori_loop(0, NSTREAM, body, 0)

    plsc.subcore_barrier()
    pltpu.sync_copy(
        acc_sh.at[pl.ds(ss * RPT, RPT)], out_hbm.at[cc, pl.ds(ss * RPT, RPT)]
    )


@functools.partial(
    pl.kernel,
    out_type=jax.ShapeDtypeStruct((NC, N_PAD, HP), jnp.float32),
    mesh=_MESH,
    scratch_types=[
        pltpu.VMEM((NSTREAM, CH), jnp.int32),
        pltpu.VMEM((NSTREAM, CH), jnp.int32),
        pltpu.VMEM((CH, HP), jnp.float32),
        pltpu.VMEM_SHARED((N_PAD, HP), jnp.float32),
        pltpu.SemaphoreType.DMA,
    ],
    compiler_params=pltpu.CompilerParams(use_tc_tiling_on_sc=False),
)
def _agg_sc(z_hbm, row_hbm, col_hbm, zrs_hbm, out_hbm, row_v, col_v, buf, acc_sh, sem):
    cc = lax.axis_index("c")
    ss = lax.axis_index("s")
    wid = ss * NC + cc

    pltpu.sync_copy(zrs_hbm.at[pl.ds(ss * RPT, RPT)], acc_sh.at[pl.ds(ss * RPT, RPT)])
    pltpu.sync_copy(row_hbm.at[wid], row_v)
    pltpu.sync_copy(col_hbm.at[wid], col_v)
    plsc.subcore_barrier()

    def body(j, carry):
        pltpu.async_copy(z_hbm.at[row_v.at[j]], buf, sem).wait()
        pltpu.sync_copy(buf, acc_sh.at[col_v.at[j]], add=True)
        return carry

    lax.fori_loop(0, NSTREAM, body, 0)

    plsc.subcore_barrier()
    pltpu.sync_copy(
        acc_sh.at[pl.ds(ss * RPT, RPT)], out_hbm.at[cc, pl.ds(ss * RPT, RPT)]
    )


def _tc1_body(degp, x, w1, z1, dinv):
    deg = (degp[0] + degp[1])[:, 0:1] + 1.0
    di = lax.rsqrt(deg)
    h = jnp.dot(x[...], w1[...], preferred_element_type=jnp.float32)
    z1[...] = di * h
    dinv[...] = di


def _tc_mid_body(aggp, z, dinv, b, w, zout):
    di = dinv[...]
    xn = jnp.maximum(di * (aggp[0] + aggp[1] + z[...]) + b[...], 0.0)
    h = jnp.dot(xn, w[...], preferred_element_type=jnp.float32)
    zout[...] = di * h


def _tc4_body(aggp, z, dinv, b, lw1, lb1, lw2, lb2, out):
    di = dinv[...]
    xn = jnp.maximum(di * (aggp[0] + aggp[1] + z[...]) + b[...], 0.0)
    ridx = lax.broadcasted_iota(jnp.int32, (N_PAD, HP), 0)
    xn = jnp.where(ridx < N, xn, -jnp.inf)
    g = jnp.max(xn, axis=0, keepdims=True)
    o1 = jnp.maximum(
        jnp.dot(g, lw1[...], preferred_element_type=jnp.float32) + lb1[...], 0.0
    )
    out[...] = jnp.dot(o1, lw2[...], preferred_element_type=jnp.float32) + lb2[...]


_tc1 = pl.pallas_call(
    _tc1_body,
    out_shape=[
        jax.ShapeDtypeStruct((N_PAD, HP), jnp.float32),
        jax.ShapeDtypeStruct((N_PAD, 1), jnp.float32),
    ],
)

_tc_mid = pl.pallas_call(
    _tc_mid_body,
    out_shape=jax.ShapeDtypeStruct((N_PAD, HP), jnp.float32),
)

_tc4 = pl.pallas_call(
    _tc4_body,
    out_shape=jax.ShapeDtypeStruct((1, 128), jnp.float32),
)


def kernel(x, edge_index, W1, b1, W2, b2, W3, b3, lW1, lb1, lW2, lb2):
    f32 = jnp.float32
    row = edge_index[0].astype(jnp.int32)
    col = edge_index[1].astype(jnp.int32)
    pad_e = E_PAD - E
    row_p = jnp.concatenate([row, jnp.full((pad_e,), N_PAD - 2, jnp.int32)])
    col_p = jnp.concatenate([col, jnp.full((pad_e,), N_PAD - 1, jnp.int32)])
    row3 = row_p.reshape(NW, NSTREAM, CH)
    col3 = col_p.reshape(NW, NSTREAM, CH)

    x_pad = jnp.pad(x.astype(f32), ((0, N_PAD - N), (0, 0)))
    W1p = jnp.pad(W1, ((0, 0), (0, HP - H)))
    W2p = jnp.pad(W2, ((0, HP - H), (0, HP - H)))
    W3p = jnp.pad(W3, ((0, HP - H), (0, HP - H)))
    b1p = jnp.pad(b1, (0, HP - H)).reshape(1, HP)
    b2p = jnp.pad(b2, (0, HP - H)).reshape(1, HP)
    b3p = jnp.pad(b3, (0, HP - H)).reshape(1, HP)
    lW1p = jnp.pad(lW1, ((0, HP - H), (0, 128 - P)))
    lb1p = jnp.pad(lb1, (0, 128 - P)).reshape(1, 128)
    lW2p = jnp.pad(lW2, ((0, 128 - P), (0, 128 - C)))
    lb2p = jnp.pad(lb2, (0, 128 - C)).reshape(1, 128)
    zrs = jnp.zeros((N_PAD, HP), f32)
    zrs8 = jnp.zeros((N_PAD, DW), f32)
    ones8 = jnp.ones((CH, DW), f32)

    degp = _deg_sc(col3, ones8, zrs8)
    z1, dinv = _tc1(degp, x_pad, W1p)
    agg1 = _agg_sc(z1, row3, col3, zrs)
    z2 = _tc_mid(agg1, z1, dinv, b1p, W2p)
    agg2 = _agg_sc(z2, row3, col3, zrs)
    z3 = _tc_mid(agg2, z2, dinv, b2p, W3p)
    agg3 = _agg_sc(z3, row3, col3, zrs)
    outp = _tc4(agg3, z3, dinv, b3p, lW1p, lb1p, lW2p, lb2p)
    return outp[:, :C]


# speedup vs baseline: 22.0576x; 1.2339x over previous
import functools

import jax
import jax.numpy as jnp
from jax import lax
from jax.experimental import pallas as pl
from jax.experimental.pallas import tpu as pltpu
from jax.experimental.pallas import tpu_sc as plsc

N = 10000
E = 320000
F_IN = 128
H = 30
P = 10
C = 2

NC, NS, L = 2, 16, 16
NW = NC * NS
N_PAD = 10240
HP = 32
E_PAD = 327680
EPW = E_PAD // NW
CH = 128
NSTREAM = EPW // CH
RPT = N_PAD // NS

_MESH = plsc.VectorSubcoreMesh(
    core_axis_name="c", subcore_axis_name="s", num_cores=NC, num_subcores=NS
)


DW = 8


@functools.partial(
    pl.kernel,
    out_type=jax.ShapeDtypeStruct((NC, N_PAD, DW), jnp.float32),
    mesh=_MESH,
    scratch_types=[
        pltpu.VMEM((NSTREAM, CH), jnp.int32),
        pltpu.VMEM((CH, DW), jnp.float32),
        pltpu.VMEM_SHARED((N_PAD, DW), jnp.float32),
    ],
    compiler_params=pltpu.CompilerParams(use_tc_tiling_on_sc=False),
)
def _deg_sc(col_hbm, ones_hbm, zrs_hbm, out_hbm, col_v, ones_v, acc_sh):
    cc = lax.axis_index("c")
    ss = lax.axis_index("s")
    wid = ss * NC + cc

    pltpu.sync_copy(zrs_hbm.at[pl.ds(ss * RPT, RPT)], acc_sh.at[pl.ds(ss * RPT, RPT)])
    pltpu.sync_copy(ones_hbm, ones_v)
    pltpu.sync_copy(col_hbm.at[wid], col_v)
    plsc.subcore_barrier()

    def body(j, carry):
        pltpu.sync_copy(ones_v, acc_sh.at[col_v.at[j]], add=True)
        return carry

    lax.fori_loop(0, NSTREAM, body, 0)

    plsc.subcore_barrier()
    pltpu.sync_copy(
        acc_sh.at[pl.ds(ss * RPT, RPT)], out_hbm.at[cc, pl.ds(ss * RPT, RPT)]
    )


NB = 8


@functools.partial(
    pl.kernel,
    out_type=jax.ShapeDtypeStruct((NC, N_PAD, HP), jnp.float32),
    mesh=_MESH,
    scratch_types=[
        pltpu.VMEM((NSTREAM, CH), jnp.int32),
        pltpu.VMEM((NSTREAM, CH), jnp.int32),
        [pltpu.VMEM((CH, HP), jnp.float32) for _ in range(NB)],
        pltpu.VMEM_SHARED((N_PAD, HP), jnp.float32),
        [pltpu.SemaphoreType.DMA for _ in range(NB)],
    ],
    compiler_params=pltpu.CompilerParams(use_tc_tiling_on_sc=False),
)
def _agg_sc(z_hbm, row_hbm, col_hbm, zrs_hbm, out_hbm, row_v, col_v, bufs, acc_sh, sems):
    cc = lax.axis_index("c")
    ss = lax.axis_index("s")
    wid = ss * NC + cc

    pltpu.sync_copy(zrs_hbm.at[pl.ds(ss * RPT, RPT)], acc_sh.at[pl.ds(ss * RPT, RPT)])
    pltpu.sync_copy(row_hbm.at[wid], row_v)
    pltpu.sync_copy(col_hbm.at[wid], col_v)
    plsc.subcore_barrier()

    for b in range(NB):
        pltpu.async_copy(z_hbm.at[row_v.at[b]], bufs[b], sems[b])

    def body(i, carry):
        for b in range(NB):
            j = i * NB + b
            pltpu.make_async_copy(z_hbm.at[row_v.at[j]], bufs[b], sems[b]).wait()
            pltpu.sync_copy(bufs[b], acc_sh.at[col_v.at[j]], add=True)
            nj = j + NB

            @pl.when(nj < NSTREAM)
            def _():
                pltpu.async_copy(z_hbm.at[row_v.at[nj]], bufs[b], sems[b])

        return carry

    lax.fori_loop(0, NSTREAM // NB, body, 0)

    plsc.subcore_barrier()
    pltpu.sync_copy(
        acc_sh.at[pl.ds(ss * RPT, RPT)], out_hbm.at[cc, pl.ds(ss * RPT, RPT)]
    )


def _tc1_body(degp, x, w1, z1, dinv):
    deg = (degp[0] + degp[1])[:, 0:1] + 1.0
    di = lax.rsqrt(deg)
    h = jnp.dot(x[...], w1[...], preferred_element_type=jnp.float32)
    z1[...] = di * h
    dinv[...] = di


def _tc_mid_body(aggp, z, dinv, b, w, zout):
    di = dinv[...]
    xn = jnp.maximum(di * (aggp[0] + aggp[1] + z[...]) + b[...], 0.0)
    h = jnp.dot(xn, w[...], preferred_element_type=jnp.float32)
    zout[...] = di * h


def _tc4_body(aggp, z, dinv, b, lw1, lb1, lw2, lb2, out):
    di = dinv[...]
    xn = jnp.maximum(di * (aggp[0] + aggp[1] + z[...]) + b[...], 0.0)
    ridx = lax.broadcasted_iota(jnp.int32, (N_PAD, HP), 0)
    xn = jnp.where(ridx < N, xn, -jnp.inf)
    g = jnp.max(xn, axis=0, keepdims=True)
    o1 = jnp.maximum(
        jnp.dot(g, lw1[...], preferred_element_type=jnp.float32) + lb1[...], 0.0
    )
    out[...] = jnp.dot(o1, lw2[...], preferred_element_type=jnp.float32) + lb2[...]


_tc1 = pl.pallas_call(
    _tc1_body,
    out_shape=[
        jax.ShapeDtypeStruct((N_PAD, HP), jnp.float32),
        jax.ShapeDtypeStruct((N_PAD, 1), jnp.float32),
    ],
)

_tc_mid = pl.pallas_call(
    _tc_mid_body,
    out_shape=jax.ShapeDtypeStruct((N_PAD, HP), jnp.float32),
)

_tc4 = pl.pallas_call(
    _tc4_body,
    out_shape=jax.ShapeDtypeStruct((1, 128), jnp.float32),
)


def kernel(x, edge_index, W1, b1, W2, b2, W3, b3, lW1, lb1, lW2, lb2):
    f32 = jnp.float32
    row = edge_index[0].astype(jnp.int32)
    col = edge_index[1].astype(jnp.int32)
    pad_e = E_PAD - E
    row_p = jnp.concatenate([row, jnp.full((pad_e,), N_PAD - 2, jnp.int32)])
    col_p = jnp.concatenate([col, jnp.full((pad_e,), N_PAD - 1, jnp.int32)])
    row3 = row_p.reshape(NW, NSTREAM, CH)
    col3 = col_p.reshape(NW, NSTREAM, CH)

    x_pad = jnp.pad(x.astype(f32), ((0, N_PAD - N), (0, 0)))
    W1p = jnp.pad(W1, ((0, 0), (0, HP - H)))
    W2p = jnp.pad(W2, ((0, HP - H), (0, HP - H)))
    W3p = jnp.pad(W3, ((0, HP - H), (0, HP - H)))
    b1p = jnp.pad(b1, (0, HP - H)).reshape(1, HP)
    b2p = jnp.pad(b2, (0, HP - H)).reshape(1, HP)
    b3p = jnp.pad(b3, (0, HP - H)).reshape(1, HP)
    lW1p = jnp.pad(lW1, ((0, HP - H), (0, 128 - P)))
    lb1p = jnp.pad(lb1, (0, 128 - P)).reshape(1, 128)
    lW2p = jnp.pad(lW2, ((0, 128 - P), (0, 128 - C)))
    lb2p = jnp.pad(lb2, (0, 128 - C)).reshape(1, 128)
    zrs = jnp.zeros((N_PAD, HP), f32)
    zrs8 = jnp.zeros((N_PAD, DW), f32)
    ones8 = jnp.ones((CH, DW), f32)

    degp = _deg_sc(col3, ones8, zrs8)
    z1, dinv = _tc1(degp, x_pad, W1p)
    agg1 = _agg_sc(z1, row3, col3, zrs)
    z2 = _tc_mid(agg1, z1, dinv, b1p, W2p)
    agg2 = _agg_sc(z2, row3, col3, zrs)
    z3 = _tc_mid(agg2, z2, dinv, b2p, W3p)
    agg3 = _agg_sc(z3, row3, col3, zrs)
    outp = _tc4(agg3, z3, dinv, b3p, lW1p, lb1p, lW2p, lb2p)
    return outp[:, :C]


# speedup vs baseline: 46.6750x; 2.1161x over previous
import functools

import jax
import jax.numpy as jnp
from jax import lax
from jax.experimental import pallas as pl
from jax.experimental.pallas import tpu as pltpu
from jax.experimental.pallas import tpu_sc as plsc

N = 10000
E = 320000
F_IN = 128
H = 30
P = 10
C = 2

NC, NS, L = 2, 16, 16
NW = NC * NS
N_PAD = 10240
HP = 32
E_PAD = 327680
EPW = E_PAD // NW
CH = 128
NSTREAM = EPW // CH
RPT = N_PAD // NS

_MESH = plsc.VectorSubcoreMesh(
    core_axis_name="c", subcore_axis_name="s", num_cores=NC, num_subcores=NS
)


DW = 8


@functools.partial(
    pl.kernel,
    out_type=jax.ShapeDtypeStruct((NC, N_PAD, DW), jnp.float32),
    mesh=_MESH,
    scratch_types=[
        pltpu.VMEM((NSTREAM, CH), jnp.int32),
        pltpu.VMEM((CH, DW), jnp.float32),
        pltpu.VMEM_SHARED((N_PAD, DW), jnp.float32),
    ],
    compiler_params=pltpu.CompilerParams(use_tc_tiling_on_sc=False),
)
def _deg_sc(col_hbm, ones_hbm, zrs_hbm, out_hbm, col_v, ones_v, acc_sh):
    cc = lax.axis_index("c")
    ss = lax.axis_index("s")
    wid = ss * NC + cc

    pltpu.sync_copy(zrs_hbm.at[pl.ds(ss * RPT, RPT)], acc_sh.at[pl.ds(ss * RPT, RPT)])
    pltpu.sync_copy(ones_hbm, ones_v)
    pltpu.sync_copy(col_hbm.at[wid], col_v)
    plsc.subcore_barrier()

    def body(j, carry):
        pltpu.sync_copy(ones_v, acc_sh.at[col_v.at[j]], add=True)
        return carry

    lax.fori_loop(0, NSTREAM, body, 0)

    plsc.subcore_barrier()
    pltpu.sync_copy(
        acc_sh.at[pl.ds(ss * RPT, RPT)], out_hbm.at[cc, pl.ds(ss * RPT, RPT)]
    )


NB = 8


@functools.partial(
    pl.kernel,
    out_type=jax.ShapeDtypeStruct((NC, N_PAD, HP), jnp.float32),
    mesh=_MESH,
    scratch_types=[
        pltpu.VMEM((NSTREAM, CH), jnp.int32),
        pltpu.VMEM((NSTREAM, CH), jnp.int32),
        [pltpu.VMEM((CH, HP), jnp.float32) for _ in range(NB)],
        pltpu.VMEM_SHARED((N_PAD, HP), jnp.float32),
        pltpu.VMEM_SHARED((N_PAD, HP), jnp.float32),
        [pltpu.SemaphoreType.DMA for _ in range(NB)],
    ],
    compiler_params=pltpu.CompilerParams(use_tc_tiling_on_sc=False),
)
def _agg_sc(z_hbm, row_hbm, col_hbm, zrs_hbm, out_hbm, row_v, col_v, bufs, z_sh, acc_sh, sems):
    cc = lax.axis_index("c")
    ss = lax.axis_index("s")
    wid = ss * NC + cc

    pltpu.sync_copy(zrs_hbm.at[pl.ds(ss * RPT, RPT)], acc_sh.at[pl.ds(ss * RPT, RPT)])
    pltpu.sync_copy(z_hbm.at[pl.ds(ss * RPT, RPT)], z_sh.at[pl.ds(ss * RPT, RPT)])
    pltpu.sync_copy(row_hbm.at[wid], row_v)
    pltpu.sync_copy(col_hbm.at[wid], col_v)
    plsc.subcore_barrier()

    for b in range(NB):
        pltpu.async_copy(z_sh.at[row_v.at[b]], bufs[b], sems[b])

    def body(i, carry):
        for b in range(NB):
            j = i * NB + b
            pltpu.make_async_copy(z_sh.at[row_v.at[j]], bufs[b], sems[b]).wait()
            pltpu.sync_copy(bufs[b], acc_sh.at[col_v.at[j]], add=True)
            nj = j + NB

            @pl.when(nj < NSTREAM)
            def _():
                pltpu.async_copy(z_sh.at[row_v.at[nj]], bufs[b], sems[b])

        return carry

    lax.fori_loop(0, NSTREAM // NB, body, 0)

    plsc.subcore_barrier()
    pltpu.sync_copy(
        acc_sh.at[pl.ds(ss * RPT, RPT)], out_hbm.at[cc, pl.ds(ss * RPT, RPT)]
    )


def _tc1_body(degp, x, w1, z1, dinv):
    deg = (degp[0] + degp[1])[:, 0:1] + 1.0
    di = lax.rsqrt(deg)
    h = jnp.dot(x[...], w1[...], preferred_element_type=jnp.float32)
    z1[...] = di * h
    dinv[...] = di


def _tc_mid_body(aggp, z, dinv, b, w, zout):
    di = dinv[...]
    xn = jnp.maximum(di * (aggp[0] + aggp[1] + z[...]) + b[...], 0.0)
    h = jnp.dot(xn, w[...], preferred_element_type=jnp.float32)
    zout[...] = di * h


def _tc4_body(aggp, z, dinv, b, lw1, lb1, lw2, lb2, out):
    di = dinv[...]
    xn = jnp.maximum(di * (aggp[0] + aggp[1] + z[...]) + b[...], 0.0)
    ridx = lax.broadcasted_iota(jnp.int32, (N_PAD, HP), 0)
    xn = jnp.where(ridx < N, xn, -jnp.inf)
    g = jnp.max(xn, axis=0, keepdims=True)
    o1 = jnp.maximum(
        jnp.dot(g, lw1[...], preferred_element_type=jnp.float32) + lb1[...], 0.0
    )
    out[...] = jnp.dot(o1, lw2[...], preferred_element_type=jnp.float32) + lb2[...]


_tc1 = pl.pallas_call(
    _tc1_body,
    out_shape=[
        jax.ShapeDtypeStruct((N_PAD, HP), jnp.float32),
        jax.ShapeDtypeStruct((N_PAD, 1), jnp.float32),
    ],
)

_tc_mid = pl.pallas_call(
    _tc_mid_body,
    out_shape=jax.ShapeDtypeStruct((N_PAD, HP), jnp.float32),
)

_tc4 = pl.pallas_call(
    _tc4_body,
    out_shape=jax.ShapeDtypeStruct((1, 128), jnp.float32),
)


def kernel(x, edge_index, W1, b1, W2, b2, W3, b3, lW1, lb1, lW2, lb2):
    f32 = jnp.float32
    row = edge_index[0].astype(jnp.int32)
    col = edge_index[1].astype(jnp.int32)
    pad_e = E_PAD - E
    row_p = jnp.concatenate([row, jnp.full((pad_e,), N_PAD - 2, jnp.int32)])
    col_p = jnp.concatenate([col, jnp.full((pad_e,), N_PAD - 1, jnp.int32)])
    row3 = row_p.reshape(NW, NSTREAM, CH)
    col3 = col_p.reshape(NW, NSTREAM, CH)

    x_pad = jnp.pad(x.astype(f32), ((0, N_PAD - N), (0, 0)))
    W1p = jnp.pad(W1, ((0, 0), (0, HP - H)))
    W2p = jnp.pad(W2, ((0, HP - H), (0, HP - H)))
    W3p = jnp.pad(W3, ((0, HP - H), (0, HP - H)))
    b1p = jnp.pad(b1, (0, HP - H)).reshape(1, HP)
    b2p = jnp.pad(b2, (0, HP - H)).reshape(1, HP)
    b3p = jnp.pad(b3, (0, HP - H)).reshape(1, HP)
    lW1p = jnp.pad(lW1, ((0, HP - H), (0, 128 - P)))
    lb1p = jnp.pad(lb1, (0, 128 - P)).reshape(1, 128)
    lW2p = jnp.pad(lW2, ((0, 128 - P), (0, 128 - C)))
    lb2p = jnp.pad(lb2, (0, 128 - C)).reshape(1, 128)
    zrs = jnp.zeros((N_PAD, HP), f32)
    zrs8 = jnp.zeros((N_PAD, DW), f32)
    ones8 = jnp.ones((CH, DW), f32)

    degp = _deg_sc(col3, ones8, zrs8)
    z1, dinv = _tc1(degp, x_pad, W1p)
    agg1 = _agg_sc(z1, row3, col3, zrs)
    z2 = _tc_mid(agg1, z1, dinv, b1p, W2p)
    agg2 = _agg_sc(z2, row3, col3, zrs)
    z3 = _tc_mid(agg2, z2, dinv, b2p, W3p)
    agg3 = _agg_sc(z3, row3, col3, zrs)
    outp = _tc4(agg3, z3, dinv, b3p, lW1p, lb1p, lW2p, lb2p)
    return outp[:, :C]


# speedup vs baseline: 53.5342x; 1.1470x over previous
import functools

import jax
import jax.numpy as jnp
from jax import lax
from jax.experimental import pallas as pl
from jax.experimental.pallas import tpu as pltpu
from jax.experimental.pallas import tpu_sc as plsc

N = 10000
E = 320000
F_IN = 128
H = 30
P = 10
C = 2

NC, NS, L = 2, 16, 16
NW = NC * NS
HP = 32
CH = 128
NSG = E // CH
NSP = NSG + 4
SPW = NSG // NW
NEXTRA = NSG - SPW * NW
MAXS = SPW + 1
RPT = N // NS
NB = 8
MAIN = (SPW // NB - 1) * NB
DW = 8

_MESH = plsc.VectorSubcoreMesh(
    core_axis_name="c", subcore_axis_name="s", num_cores=NC, num_subcores=NS
)


def _worker_streams(wid):
    base = wid * SPW + jnp.minimum(wid, NEXTRA)
    ns = jnp.where(wid < NEXTRA, SPW + 1, SPW)
    return base, ns


@functools.partial(
    pl.kernel,
    out_type=jax.ShapeDtypeStruct((N, 128), jnp.float32),
    mesh=_MESH,
    scratch_types=[
        pltpu.VMEM((MAXS, CH), jnp.int32),
        pltpu.VMEM((CH, DW), jnp.float32),
        pltpu.VMEM_SHARED((N, DW), jnp.float32),
    ],
    compiler_params=pltpu.CompilerParams(use_tc_tiling_on_sc=False),
)
def _deg_sc(col_hbm, ones_hbm, zrs_hbm, out_hbm, col_v, ones_v, acc_sh):
    cc = lax.axis_index("c")
    ss = lax.axis_index("s")
    wid = ss * NC + cc
    base, ns = _worker_streams(wid)

    pltpu.sync_copy(zrs_hbm.at[pl.ds(ss * RPT, RPT), pl.ds(0, DW)],
                    acc_sh.at[pl.ds(ss * RPT, RPT)])
    pltpu.sync_copy(ones_hbm.at[:, pl.ds(0, DW)], ones_v)
    pltpu.sync_copy(col_hbm.at[pl.ds(base, MAXS)], col_v)
    plsc.subcore_barrier()

    def body(i, carry):
        for b in range(NB):
            j = i * NB + b
            pltpu.sync_copy(ones_v, acc_sh.at[col_v.at[j]], add=True)
        return carry

    lax.fori_loop(0, SPW // NB, body, 0)

    for j in range(SPW - SPW % NB, MAXS):

        @pl.when(j < ns)
        def _():
            pltpu.sync_copy(ones_v, acc_sh.at[col_v.at[j]], add=True)

    plsc.subcore_barrier()
    pltpu.sync_copy(
        acc_sh.at[pl.ds(ss * RPT, RPT)],
        out_hbm.at[pl.ds(ss * RPT, RPT), pl.ds(cc * DW, DW)],
    )


@functools.partial(
    pl.kernel,
    out_type=jax.ShapeDtypeStruct((N, 128), jnp.float32),
    mesh=_MESH,
    scratch_types=[
        pltpu.VMEM((MAXS, CH), jnp.int32),
        pltpu.VMEM((MAXS, CH), jnp.int32),
        [pltpu.VMEM((CH, HP), jnp.float32) for _ in range(NB)],
        pltpu.VMEM_SHARED((N, HP), jnp.float32),
        pltpu.VMEM_SHARED((N, HP), jnp.float32),
        [pltpu.SemaphoreType.DMA for _ in range(NB)],
    ],
    compiler_params=pltpu.CompilerParams(use_tc_tiling_on_sc=False),
)
def _agg_sc(z_hbm, row_hbm, col_hbm, zrs_hbm, out_hbm, row_v, col_v, bufs,
            z_sh, acc_sh, sems):
    cc = lax.axis_index("c")
    ss = lax.axis_index("s")
    wid = ss * NC + cc
    base, ns = _worker_streams(wid)

    pltpu.sync_copy(zrs_hbm.at[pl.ds(ss * RPT, RPT), pl.ds(0, HP)],
                    acc_sh.at[pl.ds(ss * RPT, RPT)])
    pltpu.sync_copy(z_hbm.at[pl.ds(ss * RPT, RPT), pl.ds(0, HP)],
                    z_sh.at[pl.ds(ss * RPT, RPT)])
    pltpu.sync_copy(row_hbm.at[pl.ds(base, MAXS)], row_v)
    pltpu.sync_copy(col_hbm.at[pl.ds(base, MAXS)], col_v)
    plsc.subcore_barrier()

    for b in range(NB):
        pltpu.async_copy(z_sh.at[row_v.at[b]], bufs[b], sems[b])

    def body(i, carry):
        for b in range(NB):
            j = i * NB + b
            pltpu.make_async_copy(z_sh.at[row_v.at[j]], bufs[b], sems[b]).wait()
            pltpu.sync_copy(bufs[b], acc_sh.at[col_v.at[j]], add=True)
            pltpu.async_copy(z_sh.at[row_v.at[j + NB]], bufs[b], sems[b])
        return carry

    lax.fori_loop(0, MAIN // NB, body, 0)

    for j in range(MAIN, MAXS):
        b = j % NB

        @pl.when(j < ns)
        def _():
            pltpu.make_async_copy(z_sh.at[row_v.at[j]], bufs[b], sems[b]).wait()
            pltpu.sync_copy(bufs[b], acc_sh.at[col_v.at[j]], add=True)

        if j + NB < MAXS:

            @pl.when(j + NB < ns)
            def _():
                pltpu.async_copy(z_sh.at[row_v.at[j + NB]], bufs[b], sems[b])

    plsc.subcore_barrier()
    pltpu.sync_copy(
        acc_sh.at[pl.ds(ss * RPT, RPT)],
        out_hbm.at[pl.ds(ss * RPT, RPT), pl.ds(cc * HP, HP)],
    )


def _pack_z(z32, di):
    return jnp.concatenate([z32, di, jnp.zeros((N, 128 - HP - 1), jnp.float32)],
                           axis=1)


def _tc1_body(degp, x, w1, z1):
    dp = degp[...]
    di = lax.rsqrt(dp[:, 0:1] + dp[:, DW:DW + 1] + 1.0)
    h = jnp.dot(x[...], w1[...], preferred_element_type=jnp.float32)
    z1[...] = _pack_z(di * h, di)


def _tc_mid_body(aggp, z, b, w, zout):
    zf = z[...]
    di = zf[:, HP:HP + 1]
    a = aggp[...]
    xn = jnp.maximum(di * (a[:, :HP] + a[:, HP:2 * HP] + zf[:, :HP]) + b[...], 0.0)
    h = jnp.dot(xn, w[...], preferred_element_type=jnp.float32)
    zout[...] = _pack_z(di * h, di)


def _tc4_body(aggp, z, b, lw1, lb1, lw2, lb2, out):
    zf = z[...]
    di = zf[:, HP:HP + 1]
    a = aggp[...]
    xn = jnp.maximum(di * (a[:, :HP] + a[:, HP:2 * HP] + zf[:, :HP]) + b[...], 0.0)
    g = jnp.max(xn, axis=0, keepdims=True)
    o1 = jnp.maximum(
        jnp.dot(g, lw1[...], preferred_element_type=jnp.float32) + lb1[...], 0.0
    )
    out[...] = jnp.dot(o1, lw2[...], preferred_element_type=jnp.float32) + lb2[...]


_tc1 = pl.pallas_call(
    _tc1_body,
    out_shape=jax.ShapeDtypeStruct((N, 128), jnp.float32),
)

_tc_mid = pl.pallas_call(
    _tc_mid_body,
    out_shape=jax.ShapeDtypeStruct((N, 128), jnp.float32),
)

_tc4 = pl.pallas_call(
    _tc4_body,
    out_shape=jax.ShapeDtypeStruct((1, 128), jnp.float32),
)


def kernel(x, edge_index, W1, b1, W2, b2, W3, b3, lW1, lb1, lW2, lb2):
    f32 = jnp.float32
    row2d = jnp.pad(edge_index[0].astype(jnp.int32).reshape(NSG, CH),
                    ((0, NSP - NSG), (0, 0)))
    col2d = jnp.pad(edge_index[1].astype(jnp.int32).reshape(NSG, CH),
                    ((0, NSP - NSG), (0, 0)))

    W1p = jnp.pad(W1, ((0, 0), (0, HP - H)))
    W2p = jnp.pad(W2, ((0, HP - H), (0, HP - H)))
    W3p = jnp.pad(W3, ((0, HP - H), (0, HP - H)))
    b1p = jnp.pad(b1, (0, HP - H)).reshape(1, HP)
    b2p = jnp.pad(b2, (0, HP - H)).reshape(1, HP)
    b3p = jnp.pad(b3, (0, HP - H)).reshape(1, HP)
    lW1p = jnp.pad(lW1, ((0, HP - H), (0, 128 - P)))
    lb1p = jnp.pad(lb1, (0, 128 - P)).reshape(1, 128)
    lW2p = jnp.pad(lW2, ((0, 128 - P), (0, 128 - C)))
    lb2p = jnp.pad(lb2, (0, 128 - C)).reshape(1, 128)
    zrs = jnp.zeros((N, 128), f32)
    ones128 = jnp.ones((CH, 128), f32)

    degp = _deg_sc(col2d, ones128, zrs)
    z1 = _tc1(degp, x, W1p)
    agg1 = _agg_sc(z1, row2d, col2d, zrs)
    z2 = _tc_mid(agg1, z1, b1p, W2p)
    agg2 = _agg_sc(z2, row2d, col2d, zrs)
    z3 = _tc_mid(agg2, z2, b2p, W3p)
    agg3 = _agg_sc(z3, row2d, col2d, zrs)
    outp = _tc4(agg3, z3, b3p, lW1p, lb1p, lW2p, lb2p)
    return outp[:, :C]
